# baseline (speedup 1.0000x reference)
import jax
import jax.numpy as jnp
from jax.experimental import pallas as pl
from jax.experimental.pallas import tpu as pltpu

N = 10000
F = 128
NCLASS = 40
TILE_A = 400
TILE_B = 1000


def _body_a(x_ref, adj_ref, W_ref, b_ref, adjbf_ref, h1_ref, z_ref):
    i = pl.program_id(0)

    @pl.when(i == 0)
    def _():
        z_ref[...] = jnp.dot(x_ref[...].astype(jnp.bfloat16),
                             W_ref[...].astype(jnp.bfloat16),
                             preferred_element_type=jnp.float32
                             ).astype(jnp.bfloat16)

    a_bf = adj_ref[...].astype(jnp.bfloat16)
    adjbf_ref[...] = a_bf
    acc = jnp.dot(a_bf, z_ref[...], preferred_element_type=jnp.float32)
    h1_ref[...] = jax.nn.relu(acc + b_ref[0, :][None, :])


def _body_b(h1_ref, adj_ref, W_ref, b_ref, out_ref, h_ref, z_ref):
    l = pl.program_id(0)
    i = pl.program_id(1)

    @pl.when((l == 0) & (i == 0))
    def _():
        h_ref[...] = h1_ref[...]

    @pl.when(i == 0)
    def _():
        z_ref[...] = jnp.dot(h_ref[...].astype(jnp.bfloat16),
                             W_ref[0].astype(jnp.bfloat16),
                             preferred_element_type=jnp.float32
                             ).astype(jnp.bfloat16)

    acc = jnp.dot(adj_ref[...], z_ref[...], preferred_element_type=jnp.float32)
    b = b_ref[0, 0, :]

    @pl.when(l < 2)
    def _():
        rows = pl.ds(i * TILE_B, TILE_B)
        hv = jax.nn.relu(acc + b[None, :]) + h_ref[rows, :]
        h_ref[rows, :] = hv
        out_ref[...] = hv

    @pl.when(l == 2)
    def _():
        o = acc + b[None, :]
        mask = jax.lax.broadcasted_iota(jnp.int32, o.shape, 1) < NCLASS
        om = jnp.where(mask, o, jnp.float32(-1e30))
        m = jnp.max(om, axis=1, keepdims=True)
        e = jnp.where(mask, jnp.exp(o - m), 0.0)
        lse = jnp.log(jnp.sum(e, axis=1, keepdims=True)) + m
        out_ref[...] = o - lse


def kernel(x, adj, W_in, b_in, W_res, b_res, W_out, b_out):
    adj_bf, h1 = pl.pallas_call(
        _body_a,
        grid=(N // TILE_A,),
        in_specs=[
            pl.BlockSpec((N, F), lambda i: (0, 0)),
            pl.BlockSpec((TILE_A, N), lambda i: (i, 0)),
            pl.BlockSpec((F, F), lambda i: (0, 0)),
            pl.BlockSpec((1, F), lambda i: (0, 0)),
        ],
        out_specs=[
            pl.BlockSpec((TILE_A, N), lambda i: (i, 0)),
            pl.BlockSpec((TILE_A, F), lambda i: (i, 0)),
        ],
        out_shape=[
            jax.ShapeDtypeStruct((N, N), jnp.bfloat16),
            jax.ShapeDtypeStruct((N, F), jnp.float32),
        ],
        scratch_shapes=[pltpu.VMEM((N, F), jnp.bfloat16)],
        compiler_params=pltpu.CompilerParams(
            dimension_semantics=("arbitrary",),
        ),
    )(x, adj, W_in, b_in[None, :])

    W_pad = jnp.zeros((F, F), W_out.dtype).at[:, :NCLASS].set(W_out)
    b_pad = jnp.zeros((F,), b_out.dtype).at[:NCLASS].set(b_out)
    W_all = jnp.stack([W_res[0], W_res[1], W_pad])
    b_all = jnp.stack([b_res[0], b_res[1], b_pad])[:, None, :]

    out_full = pl.pallas_call(
        _body_b,
        grid=(3, N // TILE_B),
        in_specs=[
            pl.BlockSpec((N, F), lambda l, i: (0, 0)),
            pl.BlockSpec((TILE_B, N), lambda l, i: (i, 0)),
            pl.BlockSpec((1, F, F), lambda l, i: (l, 0, 0)),
            pl.BlockSpec((1, 1, F), lambda l, i: (l, 0, 0)),
        ],
        out_specs=pl.BlockSpec((TILE_B, F),
                               lambda l, i: (l * (N // TILE_B) + i, 0)),
        out_shape=jax.ShapeDtypeStruct((3 * N, F), jnp.float32),
        scratch_shapes=[
            pltpu.VMEM((N, F), jnp.float32),
            pltpu.VMEM((N, F), jnp.bfloat16),
        ],
        compiler_params=pltpu.CompilerParams(
            dimension_semantics=("arbitrary", "arbitrary"),
        ),
    )(h1, adj_bf, W_all, b_all)
    return out_full[2 * N:, :NCLASS]


# speedup vs baseline: 1.0064x; 1.0064x over previous
import jax
import jax.numpy as jnp
from jax.experimental import pallas as pl
from jax.experimental.pallas import tpu as pltpu

N = 10000
F = 128
NCLASS = 40
TILE_A = 400
TILE_B = 1000


def _body_a(x_ref, adj_ref, W_ref, b_ref, adjbf_ref, h1_ref, z_ref):
    i = pl.program_id(0)

    @pl.when(i == 0)
    def _():
        z_ref[...] = jnp.dot(x_ref[...].astype(jnp.bfloat16),
                             W_ref[...].astype(jnp.bfloat16),
                             preferred_element_type=jnp.float32
                             ).astype(jnp.bfloat16)

    a_bf = adj_ref[...].astype(jnp.bfloat16)
    adjbf_ref[...] = a_bf
    acc = jnp.dot(a_bf, z_ref[...], preferred_element_type=jnp.float32)
    h1_ref[...] = jax.nn.relu(acc + b_ref[0, :][None, :])


def _body_b(h1_ref, adj_ref, W_ref, b_ref, out_ref, h_ref, z_ref):
    l = pl.program_id(0)
    i = pl.program_id(1)

    @pl.when((i == 0) & (l == 0))
    def _():
        z_ref[...] = jnp.dot(h1_ref[...].astype(jnp.bfloat16),
                             W_ref[0].astype(jnp.bfloat16),
                             preferred_element_type=jnp.float32
                             ).astype(jnp.bfloat16)

    @pl.when((i == 0) & (l > 0))
    def _():
        z_ref[...] = jnp.dot(h_ref[...].astype(jnp.bfloat16),
                             W_ref[0].astype(jnp.bfloat16),
                             preferred_element_type=jnp.float32
                             ).astype(jnp.bfloat16)

    acc = jnp.dot(adj_ref[...], z_ref[...], preferred_element_type=jnp.float32)
    b = b_ref[0, 0, :]
    rows = pl.ds(i * TILE_B, TILE_B)

    @pl.when(l == 0)
    def _():
        h_ref[rows, :] = jax.nn.relu(acc + b[None, :]) + h1_ref[rows, :]

    @pl.when(l == 1)
    def _():
        h_ref[rows, :] = jax.nn.relu(acc + b[None, :]) + h_ref[rows, :]

    @pl.when(l == 2)
    def _():
        o = acc + b[None, :]
        mask = jax.lax.broadcasted_iota(jnp.int32, o.shape, 1) < NCLASS
        om = jnp.where(mask, o, jnp.float32(-1e30))
        m = jnp.max(om, axis=1, keepdims=True)
        e = jnp.where(mask, jnp.exp(o - m), 0.0)
        lse = jnp.log(jnp.sum(e, axis=1, keepdims=True)) + m
        out_ref[...] = (o - lse)[:, :NCLASS]


def kernel(x, adj, W_in, b_in, W_res, b_res, W_out, b_out):
    adj_bf, h1 = pl.pallas_call(
        _body_a,
        grid=(N // TILE_A,),
        in_specs=[
            pl.BlockSpec((N, F), lambda i: (0, 0)),
            pl.BlockSpec((TILE_A, N), lambda i: (i, 0)),
            pl.BlockSpec((F, F), lambda i: (0, 0)),
            pl.BlockSpec((1, F), lambda i: (0, 0)),
        ],
        out_specs=[
            pl.BlockSpec((TILE_A, N), lambda i: (i, 0)),
            pl.BlockSpec((TILE_A, F), lambda i: (i, 0)),
        ],
        out_shape=[
            jax.ShapeDtypeStruct((N, N), jnp.bfloat16),
            jax.ShapeDtypeStruct((N, F), jnp.float32),
        ],
        scratch_shapes=[pltpu.VMEM((N, F), jnp.bfloat16)],
        compiler_params=pltpu.CompilerParams(
            dimension_semantics=("arbitrary",),
        ),
    )(x, adj, W_in, b_in[None, :])

    W_pad = jnp.zeros((F, F), W_out.dtype).at[:, :NCLASS].set(W_out)
    b_pad = jnp.zeros((F,), b_out.dtype).at[:NCLASS].set(b_out)
    W_all = jnp.stack([W_res[0], W_res[1], W_pad])
    b_all = jnp.stack([b_res[0], b_res[1], b_pad])[:, None, :]

    out_full = pl.pallas_call(
        _body_b,
        grid=(3, N // TILE_B),
        in_specs=[
            pl.BlockSpec((N, F), lambda l, i: (0, 0)),
            pl.BlockSpec((TILE_B, N), lambda l, i: (i, 0)),
            pl.BlockSpec((1, F, F), lambda l, i: (l, 0, 0)),
            pl.BlockSpec((1, 1, F), lambda l, i: (l, 0, 0)),
        ],
        out_specs=pl.BlockSpec((TILE_B, NCLASS),
                               lambda l, i: (jnp.where(l == 2, i, 0), 0)),
        out_shape=jax.ShapeDtypeStruct((N, NCLASS), jnp.float32),
        scratch_shapes=[
            pltpu.VMEM((N, F), jnp.float32),
            pltpu.VMEM((N, F), jnp.bfloat16),
        ],
        compiler_params=pltpu.CompilerParams(
            dimension_semantics=("arbitrary", "arbitrary"),
        ),
    )(h1, adj_bf, W_all, b_all)
    return out_full


# speedup vs baseline: 1.0124x; 1.0060x over previous
import jax
import jax.numpy as jnp
from jax.experimental import pallas as pl
from jax.experimental.pallas import tpu as pltpu

N = 10000
F = 128
NCLASS = 40
TILE_A = 400
TILE_B = 1000


def _body_a(x_ref, adj_ref, Win_ref, Wr0_ref, b_ref,
            adjbf_ref, h1_ref, z1_ref, z0_ref):
    i = pl.program_id(0)

    @pl.when(i == 0)
    def _():
        z0_ref[...] = jnp.dot(x_ref[...].astype(jnp.bfloat16),
                              Win_ref[...].astype(jnp.bfloat16),
                              preferred_element_type=jnp.float32
                              ).astype(jnp.bfloat16)

    a_bf = adj_ref[...].astype(jnp.bfloat16)
    adjbf_ref[...] = a_bf
    acc = jnp.dot(a_bf, z0_ref[...], preferred_element_type=jnp.float32)
    hv = jax.nn.relu(acc + b_ref[0, :][None, :]).astype(jnp.bfloat16)
    h1_ref[...] = hv
    z1_ref[...] = jnp.dot(hv, Wr0_ref[...].astype(jnp.bfloat16),
                          preferred_element_type=jnp.float32
                          ).astype(jnp.bfloat16)


def _body_b(h1_ref, z1_ref, adj_ref, W_ref, b_ref, out_ref,
            h_ref, za_ref, zb_ref):
    l = pl.program_id(0)
    i = pl.program_id(1)
    b = b_ref[0, 0, :]
    rows = pl.ds(i * TILE_B, TILE_B)
    W_bf = W_ref[0].astype(jnp.bfloat16)

    @pl.when(l == 0)
    def _():
        acc = jnp.dot(adj_ref[...], z1_ref[...],
                      preferred_element_type=jnp.float32)
        hv = (jax.nn.relu(acc + b[None, :])
              + h1_ref[rows, :].astype(jnp.float32)).astype(jnp.bfloat16)
        h_ref[rows, :] = hv
        za_ref[rows, :] = jnp.dot(hv, W_bf,
                                  preferred_element_type=jnp.float32
                                  ).astype(jnp.bfloat16)

    @pl.when(l == 1)
    def _():
        acc = jnp.dot(adj_ref[...], za_ref[...],
                      preferred_element_type=jnp.float32)
        hv = (jax.nn.relu(acc + b[None, :])
              + h_ref[rows, :].astype(jnp.float32)).astype(jnp.bfloat16)
        zb_ref[rows, :] = jnp.dot(hv, W_bf,
                                  preferred_element_type=jnp.float32
                                  ).astype(jnp.bfloat16)

    @pl.when(l == 2)
    def _():
        acc = jnp.dot(adj_ref[...], zb_ref[...],
                      preferred_element_type=jnp.float32)
        o = acc + b[None, :]
        mask = jax.lax.broadcasted_iota(jnp.int32, o.shape, 1) < NCLASS
        om = jnp.where(mask, o, jnp.float32(-1e30))
        m = jnp.max(om, axis=1, keepdims=True)
        e = jnp.where(mask, jnp.exp(o - m), 0.0)
        lse = jnp.log(jnp.sum(e, axis=1, keepdims=True)) + m
        out_ref[...] = (o - lse)[:, :NCLASS]


def kernel(x, adj, W_in, b_in, W_res, b_res, W_out, b_out):
    adj_bf, h1, z1 = pl.pallas_call(
        _body_a,
        grid=(N // TILE_A,),
        in_specs=[
            pl.BlockSpec((N, F), lambda i: (0, 0)),
            pl.BlockSpec((TILE_A, N), lambda i: (i, 0)),
            pl.BlockSpec((F, F), lambda i: (0, 0)),
            pl.BlockSpec((F, F), lambda i: (0, 0)),
            pl.BlockSpec((1, F), lambda i: (0, 0)),
        ],
        out_specs=[
            pl.BlockSpec((TILE_A, N), lambda i: (i, 0)),
            pl.BlockSpec((TILE_A, F), lambda i: (i, 0)),
            pl.BlockSpec((TILE_A, F), lambda i: (i, 0)),
        ],
        out_shape=[
            jax.ShapeDtypeStruct((N, N), jnp.bfloat16),
            jax.ShapeDtypeStruct((N, F), jnp.bfloat16),
            jax.ShapeDtypeStruct((N, F), jnp.bfloat16),
        ],
        scratch_shapes=[pltpu.VMEM((N, F), jnp.bfloat16)],
        compiler_params=pltpu.CompilerParams(
            dimension_semantics=("arbitrary",),
        ),
    )(x, adj, W_in, W_res[0], b_in[None, :])

    W_pad = jnp.zeros((F, F), W_out.dtype).at[:, :NCLASS].set(W_out)
    b_pad = jnp.zeros((F,), b_out.dtype).at[:NCLASS].set(b_out)
    W_all = jnp.stack([W_res[1], W_pad])
    b_all = jnp.stack([b_res[0], b_res[1], b_pad])[:, None, :]

    out = pl.pallas_call(
        _body_b,
        grid=(3, N // TILE_B),
        in_specs=[
            pl.BlockSpec((N, F), lambda l, i: (0, 0)),
            pl.BlockSpec((N, F), lambda l, i: (0, 0)),
            pl.BlockSpec((TILE_B, N), lambda l, i: (i, 0)),
            pl.BlockSpec((1, F, F), lambda l, i: (jnp.minimum(l, 1), 0, 0)),
            pl.BlockSpec((1, 1, F), lambda l, i: (l, 0, 0)),
        ],
        out_specs=pl.BlockSpec((TILE_B, NCLASS),
                               lambda l, i: (jnp.where(l == 2, i, 0), 0)),
        out_shape=jax.ShapeDtypeStruct((N, NCLASS), jnp.float32),
        scratch_shapes=[
            pltpu.VMEM((N, F), jnp.bfloat16),
            pltpu.VMEM((N, F), jnp.bfloat16),
            pltpu.VMEM((N, F), jnp.bfloat16),
        ],
        compiler_params=pltpu.CompilerParams(
            dimension_semantics=("arbitrary", "arbitrary"),
        ),
    )(h1, z1, adj_bf, W_all, b_all)
    return out


# speedup vs baseline: 1.0208x; 1.0083x over previous
import jax
import jax.numpy as jnp
from jax.experimental import pallas as pl
from jax.experimental.pallas import tpu as pltpu

N = 10000
F = 128
NCLASS = 40
TILE_A = 400
TILE_B = 1000


def _body_a(x_ref, adj_ref, Win_ref, Wr0_ref, b_ref,
            adjbf_ref, h1_ref, z1_ref, z0_ref):
    i = pl.program_id(0)

    @pl.when(i == 0)
    def _():
        z0_ref[...] = jnp.dot(x_ref[...].astype(jnp.bfloat16),
                              Win_ref[...].astype(jnp.bfloat16),
                              preferred_element_type=jnp.float32
                              ).astype(jnp.bfloat16)

    a_bf = adj_ref[...].astype(jnp.bfloat16)
    adjbf_ref[...] = a_bf
    acc = jnp.dot(a_bf, z0_ref[...], preferred_element_type=jnp.float32)
    hv = jax.nn.relu(acc + b_ref[0, :][None, :]).astype(jnp.bfloat16)
    h1_ref[...] = hv
    z1_ref[...] = jnp.dot(hv, Wr0_ref[...].astype(jnp.bfloat16),
                          preferred_element_type=jnp.float32
                          ).astype(jnp.bfloat16)


def _body_b(h1_ref, z1_ref, adj_ref, Wr1_ref, Wout_ref, br_ref, bout_ref,
            out_ref, h_ref, za_ref, zb_ref):
    l = pl.program_id(0)
    i = pl.program_id(1)
    rows = pl.ds(i * TILE_B, TILE_B)

    @pl.when(l == 0)
    def _():
        acc = jnp.dot(adj_ref[...], z1_ref[...],
                      preferred_element_type=jnp.float32)
        hv = (jax.nn.relu(acc + br_ref[0, 0, :][None, :])
              + h1_ref[rows, :].astype(jnp.float32)).astype(jnp.bfloat16)
        h_ref[rows, :] = hv
        za_ref[rows, :] = jnp.dot(hv, Wr1_ref[0].astype(jnp.bfloat16),
                                  preferred_element_type=jnp.float32
                                  ).astype(jnp.bfloat16)

    @pl.when(l == 1)
    def _():
        acc = jnp.dot(adj_ref[...], za_ref[...],
                      preferred_element_type=jnp.float32)
        hv = (jax.nn.relu(acc + br_ref[0, 0, :][None, :])
              + h_ref[rows, :].astype(jnp.float32)).astype(jnp.bfloat16)
        zb_ref[rows, :] = jnp.dot(hv, Wout_ref[...].astype(jnp.bfloat16),
                                  preferred_element_type=jnp.float32
                                  ).astype(jnp.bfloat16)

    @pl.when(l == 2)
    def _():
        acc = jnp.dot(adj_ref[...], zb_ref[...],
                      preferred_element_type=jnp.float32)
        o = acc + bout_ref[0, :][None, :]
        m = jnp.max(o, axis=1, keepdims=True)
        e = jnp.exp(o - m)
        lse = jnp.log(jnp.sum(e, axis=1, keepdims=True)) + m
        out_ref[...] = o - lse


def kernel(x, adj, W_in, b_in, W_res, b_res, W_out, b_out):
    adj_bf, h1, z1 = pl.pallas_call(
        _body_a,
        grid=(N // TILE_A,),
        in_specs=[
            pl.BlockSpec((N, F), lambda i: (0, 0)),
            pl.BlockSpec((TILE_A, N), lambda i: (i, 0)),
            pl.BlockSpec((F, F), lambda i: (0, 0)),
            pl.BlockSpec((F, F), lambda i: (0, 0)),
            pl.BlockSpec((1, F), lambda i: (0, 0)),
        ],
        out_specs=[
            pl.BlockSpec((TILE_A, N), lambda i: (i, 0)),
            pl.BlockSpec((TILE_A, F), lambda i: (i, 0)),
            pl.BlockSpec((TILE_A, F), lambda i: (i, 0)),
        ],
        out_shape=[
            jax.ShapeDtypeStruct((N, N), jnp.bfloat16),
            jax.ShapeDtypeStruct((N, F), jnp.bfloat16),
            jax.ShapeDtypeStruct((N, F), jnp.bfloat16),
        ],
        scratch_shapes=[pltpu.VMEM((N, F), jnp.bfloat16)],
        compiler_params=pltpu.CompilerParams(
            dimension_semantics=("arbitrary",),
        ),
    )(x, adj, W_in, W_res[0], b_in[None, :])

    out = pl.pallas_call(
        _body_b,
        grid=(3, N // TILE_B),
        in_specs=[
            pl.BlockSpec((N, F), lambda l, i: (0, 0)),
            pl.BlockSpec((N, F), lambda l, i: (0, 0)),
            pl.BlockSpec((TILE_B, N), lambda l, i: (i, 0)),
            pl.BlockSpec((1, F, F), lambda l, i: (1, 0, 0)),
            pl.BlockSpec((F, NCLASS), lambda l, i: (0, 0)),
            pl.BlockSpec((1, 1, F), lambda l, i: (jnp.minimum(l, 1), 0, 0)),
            pl.BlockSpec((1, NCLASS), lambda l, i: (0, 0)),
        ],
        out_specs=pl.BlockSpec((TILE_B, NCLASS),
                               lambda l, i: (jnp.where(l == 2, i, 0), 0)),
        out_shape=jax.ShapeDtypeStruct((N, NCLASS), jnp.float32),
        scratch_shapes=[
            pltpu.VMEM((N, F), jnp.bfloat16),
            pltpu.VMEM((N, F), jnp.bfloat16),
            pltpu.VMEM((N, NCLASS), jnp.bfloat16),
        ],
        compiler_params=pltpu.CompilerParams(
            dimension_semantics=("arbitrary", "arbitrary"),
        ),
    )(h1, z1, adj_bf, W_res, W_out, b_res[:, None, :], b_out[None, :])
    return out
